# Initial kernel scaffold; baseline (speedup 1.0000x reference)
import jax
import jax.numpy as jnp
from jax.experimental import pallas as pl


def kernel(y_hat, efs_time, efs):
    raise NotImplementedError("write your pallas kernel here")


# speedup vs baseline: 4900.7333x; 4900.7333x over previous
import functools

import jax
import jax.numpy as jnp
from jax import lax
from jax.experimental import pallas as pl
from jax.experimental.pallas import tpu as pltpu
from jax.experimental.pallas import tpu_sc as plsc

B = 4096
MARGIN = 0.5
L = 16
NC = 2
NS = 16
NW = NC * NS
ROWS_PER_W = B // NW
NCHUNK = B // L

_mesh = plsc.VectorSubcoreMesh(core_axis_name="c", subcore_axis_name="s")


@functools.partial(
    pl.kernel,
    mesh=_mesh,
    out_type=[
        jax.ShapeDtypeStruct((NW, L), jnp.float32),
        jax.ShapeDtypeStruct((NW, L), jnp.float32),
    ],
    scratch_types=[
        pltpu.VMEM((B,), jnp.float32),
        pltpu.VMEM((B,), jnp.float32),
        pltpu.VMEM((B,), jnp.float32),
        pltpu.VMEM((L,), jnp.float32),
        pltpu.VMEM((L,), jnp.float32),
    ],
)
def _pairwise_sc(p_hbm, t_hbm, e_hbm, num_hbm, den_hbm, pv, tv, ev, nv, dv):
    cid = lax.axis_index("c")
    sid = lax.axis_index("s")
    wid = sid * NC + cid

    pltpu.sync_copy(p_hbm, pv)
    pltpu.sync_copy(t_hbm, tv)
    pltpu.sync_copy(e_hbm, ev)

    lanes = lax.iota(jnp.int32, L)
    zeros = jnp.zeros((L,), jnp.float32)

    def contrib(p_i, t_i, e_i, p_j, t_j, e_j):
        lt = t_i < t_j
        gt = t_j < t_i
        d = p_i - p_j
        yd = jnp.where(lt, d, -d)
        h = jnp.maximum(MARGIN - yd, 0.0)
        m = jnp.where(lt, e_i, jnp.where(gt, e_j, e_i * e_j))
        return h * m, m

    def row_body(k, carry):
        num, den = carry
        i = wid + NW * k
        cc = i // L
        base = cc * L
        p_j = pv[pl.ds(base, L)]
        t_j = tv[pl.ds(base, L)]
        e_j = ev[pl.ds(base, L)]
        lane = jnp.full((L,), i - base, dtype=jnp.int32)
        p_i = p_j.at[lane].get(mode="promise_in_bounds")
        t_i = t_j.at[lane].get(mode="promise_in_bounds")
        e_i = e_j.at[lane].get(mode="promise_in_bounds")
        hm, m = contrib(p_i, t_i, e_i, p_j, t_j, e_j)
        tri = (base + lanes) > i
        num = num + jnp.where(tri, hm, zeros)
        den = den + jnp.where(tri, m, zeros)

        def chunk_body(c, carry2):
            num2, den2 = carry2
            b2 = c * L
            hm2, m2 = contrib(p_i, t_i, e_i,
                              pv[pl.ds(b2, L)], tv[pl.ds(b2, L)], ev[pl.ds(b2, L)])
            return num2 + hm2, den2 + m2

        return lax.fori_loop(cc + 1, NCHUNK, chunk_body, (num, den))

    num, den = lax.fori_loop(0, ROWS_PER_W, row_body, (zeros, zeros))
    nv[...] = num
    dv[...] = den
    pltpu.sync_copy(nv, num_hbm.at[wid])
    pltpu.sync_copy(dv, den_hbm.at[wid])


def _final_reduce(num_ref, den_ref, out_ref):
    s = jnp.sum(num_ref[...]) / jnp.sum(den_ref[...])
    out_ref[...] = jnp.full((1, 1), s, jnp.float32)


def kernel(y_hat, efs_time, efs):
    y_hat = jnp.squeeze(y_hat).astype(jnp.float32)
    efs_time = efs_time.astype(jnp.float32)
    efs_f = efs.astype(jnp.float32)
    num, den = _pairwise_sc(y_hat, efs_time, efs_f)
    out = pl.pallas_call(
        _final_reduce,
        out_shape=jax.ShapeDtypeStruct((1, 1), jnp.float32),
    )(num, den)
    return out[0, 0]


# speedup vs baseline: 5256.0263x; 1.0725x over previous
import functools

import jax
import jax.numpy as jnp
from jax import lax
from jax.experimental import pallas as pl
from jax.experimental.pallas import tpu as pltpu
from jax.experimental.pallas import tpu_sc as plsc

B = 4096
MARGIN = 0.5
L = 16
NC = 2
NS = 16
NW = NC * NS
ROWS_PER_W = B // NW
NCHUNK = B // L

_mesh = plsc.VectorSubcoreMesh(core_axis_name="c", subcore_axis_name="s")


@functools.partial(
    pl.kernel,
    mesh=_mesh,
    out_type=[
        jax.ShapeDtypeStruct((NW, L), jnp.float32),
        jax.ShapeDtypeStruct((NW, L), jnp.float32),
    ],
    scratch_types=[
        pltpu.VMEM((B,), jnp.float32),
        pltpu.VMEM((B,), jnp.float32),
        pltpu.VMEM((B + L,), jnp.float32),
        pltpu.VMEM((L,), jnp.float32),
        pltpu.VMEM((L,), jnp.float32),
    ],
)
def _pairwise_sc(p_hbm, t_hbm, e_hbm, num_hbm, den_hbm, pv, tv, ev, nv, dv):
    cid = lax.axis_index("c")
    sid = lax.axis_index("s")
    wid = sid * NC + cid

    pltpu.sync_copy(p_hbm, pv)
    pltpu.sync_copy(t_hbm, tv)
    pltpu.sync_copy(e_hbm, ev.at[pl.ds(0, B)])

    lanes = lax.iota(jnp.int32, L)
    zeros = jnp.zeros((L,), jnp.float32)
    ones = jnp.ones((L,), jnp.float32)

    def contrib(p_i, t_i, e_i, p_j, t_j, e_j):
        lt = t_i < t_j
        gt = t_j < t_i
        d = p_i - p_j
        yd = jnp.where(lt, d, -d)
        h = jnp.maximum(MARGIN - yd, 0.0)
        m = jnp.where(lt, e_i, jnp.where(gt, e_j, e_i * e_j))
        return h * m, m

    nv[...] = zeros
    dv[...] = zeros

    def row_body(k, dummy):
        i = wid + NW * k
        cc = i // L
        base = cc * L
        p_c = pv[pl.ds(base, L)]
        t_c = tv[pl.ds(base, L)]
        e_c = ev[pl.ds(base, L)]
        lane = jnp.full((L,), i - base, dtype=jnp.int32)
        p_i = p_c.at[lane].get(mode="promise_in_bounds")
        t_i = t_c.at[lane].get(mode="promise_in_bounds")
        e_i = e_c.at[lane].get(mode="promise_in_bounds")
        hm, m = contrib(p_i, t_i, e_i, p_c, t_c, e_c)
        tri = (base + lanes) > i
        nv[...] = nv[...] + jnp.where(tri, hm, zeros)
        dv[...] = dv[...] + jnp.where(tri, m, zeros)

        def rows_with_event(_):
            @plsc.parallel_loop(cc + 1, NCHUNK, unroll=4, carry=(zeros, zeros))
            def loop1(c, carry2):
                num2, den2 = carry2
                b2 = c * L
                p_j = pv[pl.ds(b2, L)]
                t_j = tv[pl.ds(b2, L)]
                e_j = ev[pl.ds(b2, L)]
                lt = t_i < t_j
                d = p_i - p_j
                yd = jnp.where(lt, d, -d)
                h = jnp.maximum(MARGIN - yd, 0.0)
                m = jnp.where(lt, ones, e_j)
                return num2 + h * m, den2 + m

            n1, d1 = loop1
            nv[...] = nv[...] + n1
            dv[...] = dv[...] + d1

        def rows_without_event(_):
            @plsc.parallel_loop(cc + 1, NCHUNK, unroll=4, carry=(zeros, zeros))
            def loop0(c, carry2):
                num2, den2 = carry2
                b2 = c * L
                p_j = pv[pl.ds(b2, L)]
                t_j = tv[pl.ds(b2, L)]
                e_j = ev[pl.ds(b2, L)]
                gt = t_j < t_i
                h = jnp.maximum(MARGIN + (p_i - p_j), 0.0)
                m = jnp.where(gt, e_j, zeros)
                return num2 + h * m, den2 + m

            n0, d0 = loop0
            nv[...] = nv[...] + n0
            dv[...] = dv[...] + d0

        has_event = ev[pl.ds(i, L)][0] > 0.0
        lax.cond(has_event, rows_with_event, rows_without_event, 0)
        return dummy

    lax.fori_loop(0, ROWS_PER_W, row_body, 0)
    pltpu.sync_copy(nv, num_hbm.at[wid])
    pltpu.sync_copy(dv, den_hbm.at[wid])


def _final_reduce(num_ref, den_ref, out_ref):
    s = jnp.sum(num_ref[...]) / jnp.sum(den_ref[...])
    out_ref[...] = jnp.full((1, 1), s, jnp.float32)


def kernel(y_hat, efs_time, efs):
    y_hat = jnp.squeeze(y_hat).astype(jnp.float32)
    efs_time = efs_time.astype(jnp.float32)
    efs_f = efs.astype(jnp.float32)
    num, den = _pairwise_sc(y_hat, efs_time, efs_f)
    out = pl.pallas_call(
        _final_reduce,
        out_shape=jax.ShapeDtypeStruct((1, 1), jnp.float32),
    )(num, den)
    return out[0, 0]
